# baseline (speedup 1.0000x reference)
import functools

import jax
import jax.numpy as jnp
from jax import lax
from jax.experimental import pallas as pl
from jax.experimental.pallas import tpu as pltpu
from jax.experimental.pallas import tpu_sc as plsc

_INTERPRET = False

H = 1024
I = 512
E = 8
T = 2048
TOPK = 2
TILE = 256
G_R = 23
NPAD = G_R * TILE
G_S = T // TILE
GPAD = 128

_NC = 2
_NS = 16
_NW = _NC * _NS
_TPW = T // _NW
_CH = 16
_NCHUNK = _TPW // _CH


def _router_kernel(x_ref, gw_ref, b_ref, w0_ref, w1_ref,
                   pos_ref, te_ref):
    x = x_ref[...]
    gw = gw_ref[...]
    logits = lax.dot_general(x, gw, (((1,), (1,)), ((), ())),
                             preferred_element_type=jnp.float32)
    scores = jax.nn.sigmoid(logits)
    s4c = scores + b_ref[...]
    lane = lax.broadcasted_iota(jnp.int32, (T, E), 1)
    m1 = jnp.max(s4c, axis=1, keepdims=True)
    a1 = jnp.min(jnp.where(s4c >= m1, lane, E), axis=1, keepdims=True)
    oh1 = lane == a1
    s_m = jnp.where(oh1, -jnp.inf, s4c)
    m2 = jnp.max(s_m, axis=1, keepdims=True)
    a2 = jnp.min(jnp.where(s_m >= m2, lane, E), axis=1, keepdims=True)
    oh2 = lane == a2
    w1 = jnp.sum(jnp.where(oh1, scores, 0.0), axis=1, keepdims=True)
    w2 = jnp.sum(jnp.where(oh2, scores, 0.0), axis=1, keepdims=True)
    ws = w1 + w2 + 1e-20
    w0_ref[...] = jnp.broadcast_to(w1 / ws, (T, 16))
    w1_ref[...] = jnp.broadcast_to(w2 / ws, (T, 16))

    cnt = oh1.astype(jnp.float32) + oh2.astype(jnp.float32)
    s = cnt
    sh = 1
    while sh < T:
        s = s + jnp.concatenate(
            [jnp.zeros((sh, E), jnp.float32), s[:-sh, :]], axis=0)
        sh *= 2
    excl = s - cnt
    r0 = jnp.sum(jnp.where(oh1, excl, 0.0), axis=1, keepdims=True)
    r1 = jnp.sum(jnp.where(oh2, excl, 0.0), axis=1, keepdims=True)
    counts = s[T - 1:T, :]
    nt = jnp.floor((counts + (TILE - 1)) / TILE)
    p = nt
    for k in (1, 2, 4):
        p = p + jnp.concatenate(
            [jnp.zeros((1, k), jnp.float32), p[:, :-k]], axis=1)
    tile_start = p - nt
    row_off = tile_start * TILE
    base0 = jnp.sum(jnp.where(oh1, row_off, 0.0), axis=1, keepdims=True)
    base1 = jnp.sum(jnp.where(oh2, row_off, 0.0), axis=1, keepdims=True)
    pos_ref[...] = jnp.concatenate(
        [base0 + r0, base1 + r1], axis=1).astype(jnp.int32)

    gi = lax.broadcasted_iota(jnp.int32, (GPAD, E), 0)
    ts = jnp.floor(tile_start).astype(jnp.int32)
    te = jnp.sum((gi >= ts).astype(jnp.int32), axis=1, keepdims=True) - 1
    te_ref[...] = jnp.maximum(te, 0)


def _dispatch_kernel(x_hbm, p0_hbm, p1_hbm, buf_hbm,
                     idx0_v, idx1_v, rows0_v, rows1_v,
                     semL0, semL1, semS0, semS1):
    wid = lax.axis_index("s") * _NC + lax.axis_index("c")
    pltpu.sync_copy(p0_hbm.at[pl.ds(_NCHUNK * wid, _NCHUNK)], idx0_v)
    pltpu.sync_copy(p1_hbm.at[pl.ds(_NCHUNK * wid, _NCHUNK)], idx1_v)
    rows = (rows0_v, rows1_v)
    semL = (semL0, semL1)
    semS = (semS0, semS1)
    loads = [None, None]
    scats = [None, None]

    def start_load(c):
        base = wid * _TPW + c * _CH
        loads[c % 2] = pltpu.async_copy(
            x_hbm.at[pl.ds(base, _CH)], rows[c % 2], semL[c % 2])

    start_load(0)
    for c in range(_NCHUNK):
        loads[c % 2].wait()
        if c + 1 < _NCHUNK:
            if scats[(c + 1) % 2] is not None:
                for s in scats[(c + 1) % 2]:
                    s.wait()
                scats[(c + 1) % 2] = None
            start_load(c + 1)
        scats[c % 2] = (
            pltpu.async_copy(rows[c % 2], buf_hbm.at[idx0_v.at[c]],
                             semS[c % 2]),
            pltpu.async_copy(rows[c % 2], buf_hbm.at[idx1_v.at[c]],
                             semS[c % 2]),
        )
    for pair in scats:
        if pair is not None:
            for s in pair:
                s.wait()


def _combine_kernel(ys_hbm, yr_hbm, p0_hbm, p1_hbm, w0_hbm, w1_hbm,
                    out_hbm,
                    idx0_v, idx1_v, w0_v, w1_v,
                    rowsA0_v, rowsA1_v, rowsB0_v, rowsB1_v,
                    rowsC0_v, rowsC1_v,
                    semG0, semG1, semO0, semO1):
    wid = lax.axis_index("s") * _NC + lax.axis_index("c")
    pltpu.sync_copy(p0_hbm.at[pl.ds(_NCHUNK * wid, _NCHUNK)], idx0_v)
    pltpu.sync_copy(p1_hbm.at[pl.ds(_NCHUNK * wid, _NCHUNK)], idx1_v)
    pltpu.sync_copy(w0_hbm.at[pl.ds(wid * _TPW, _TPW)], w0_v)
    pltpu.sync_copy(w1_hbm.at[pl.ds(wid * _TPW, _TPW)], w1_v)
    rowsA = (rowsA0_v, rowsA1_v)
    rowsB = (rowsB0_v, rowsB1_v)
    rowsC = (rowsC0_v, rowsC1_v)
    semG = (semG0, semG1)
    semO = (semO0, semO1)
    gaths = [None, None]
    stores = [None, None]

    def start_gathers(c):
        base = wid * _TPW + c * _CH
        k = c % 2
        gaths[k] = (
            pltpu.async_copy(yr_hbm.at[idx0_v.at[c]], rowsA[k], semG[k]),
            pltpu.async_copy(yr_hbm.at[idx1_v.at[c]], rowsB[k], semG[k]),
            pltpu.async_copy(ys_hbm.at[pl.ds(base, _CH)], rowsC[k], semG[k]),
        )

    start_gathers(0)
    for c in range(_NCHUNK):
        k = c % 2
        for gd in gaths[k]:
            gd.wait()
        if c + 1 < _NCHUNK:
            if stores[(c + 1) % 2] is not None:
                stores[(c + 1) % 2].wait()
                stores[(c + 1) % 2] = None
            start_gathers(c + 1)
        for r in range(_CH):
            t = c * _CH + r
            g0 = w0_v[t]
            g1 = w1_v[t]

            def add_body(j, carry, r=r, g0=g0, g1=g1, k=k):
                for u in range(16):
                    sl = pl.ds(j * 256 + u * 16, 16)
                    rowsC[k][r, sl] = (rowsC[k][r, sl]
                                       + g0 * rowsA[k][r, sl]
                                       + g1 * rowsB[k][r, sl])
                return carry

            lax.fori_loop(0, H // 256, add_body, 0)
        base = wid * _TPW + c * _CH
        stores[k] = pltpu.async_copy(
            rowsC[k], out_hbm.at[pl.ds(base, _CH)], semO[k])
    for st in stores:
        if st is not None:
            st.wait()


def _mlp_kernel(te_ref, x_ref, gu_ref, dn_ref, y_ref):
    xb = x_ref[...].astype(jnp.bfloat16)
    gu = gu_ref[0].astype(jnp.bfloat16)
    a = lax.dot_general(xb, gu, (((1,), (1,)), ((), ())),
                        preferred_element_type=jnp.float32)
    g = a[:, :I]
    u = a[:, I:]
    h = (g * jax.nn.sigmoid(g)) * u
    dn = dn_ref[0].astype(jnp.bfloat16)
    y = lax.dot_general(h.astype(jnp.bfloat16), dn,
                        (((1,), (1,)), ((), ())),
                        preferred_element_type=jnp.float32)
    y_ref[...] = y


def _shared_kernel(x_ref, gw_ref, uw_ref, dw_ref, y_ref):
    xb = x_ref[...].astype(jnp.bfloat16)
    g = lax.dot_general(xb, gw_ref[...].astype(jnp.bfloat16),
                        (((1,), (1,)), ((), ())),
                        preferred_element_type=jnp.float32)
    u = lax.dot_general(xb, uw_ref[...].astype(jnp.bfloat16),
                        (((1,), (1,)), ((), ())),
                        preferred_element_type=jnp.float32)
    h = (g * jax.nn.sigmoid(g)) * u
    y = lax.dot_general(h.astype(jnp.bfloat16),
                        dw_ref[...].astype(jnp.bfloat16),
                        (((1,), (1,)), ((), ())),
                        preferred_element_type=jnp.float32)
    y_ref[...] = y


def kernel(x, gate_weight, e_score_correction_bias, experts_gate_up,
           experts_down, shared_gate_w, shared_up_w, shared_down_w):
    B, S, Hd = x.shape
    x_flat = x.reshape(-1, Hd).astype(jnp.float32)

    w0b, w1b, pos, te_full = pl.pallas_call(
        _router_kernel,
        out_shape=[
            jax.ShapeDtypeStruct((T, 16), jnp.float32),
            jax.ShapeDtypeStruct((T, 16), jnp.float32),
            jax.ShapeDtypeStruct((T, TOPK), jnp.int32),
            jax.ShapeDtypeStruct((GPAD, 1), jnp.int32),
        ],
        interpret=_INTERPRET,
    )(x_flat, gate_weight.astype(jnp.float32),
      e_score_correction_bias.reshape(1, E).astype(jnp.float32))

    te = te_full[:G_R, 0]

    p0 = pos[:, 0].reshape(T // _CH, _CH)
    p1 = pos[:, 1].reshape(T // _CH, _CH)
    mesh = plsc.VectorSubcoreMesh(core_axis_name="c", subcore_axis_name="s",
                                  num_cores=_NC, num_subcores=_NS)
    buffer = pl.kernel(
        _dispatch_kernel,
        out_type=jax.ShapeDtypeStruct((NPAD, H), jnp.float32),
        mesh=mesh,
        scratch_types=[
            pltpu.VMEM((_NCHUNK, _CH), jnp.int32),
            pltpu.VMEM((_NCHUNK, _CH), jnp.int32),
            pltpu.VMEM((_CH, H), jnp.float32),
            pltpu.VMEM((_CH, H), jnp.float32),
            pltpu.SemaphoreType.DMA,
            pltpu.SemaphoreType.DMA,
            pltpu.SemaphoreType.DMA,
            pltpu.SemaphoreType.DMA,
        ],
        interpret=_INTERPRET,
    )(x_flat, p0, p1)

    y_s = pl.pallas_call(
        _shared_kernel,
        grid=(G_S,),
        in_specs=[
            pl.BlockSpec((TILE, H), lambda g: (g, 0)),
            pl.BlockSpec((I, H), lambda g: (0, 0)),
            pl.BlockSpec((I, H), lambda g: (0, 0)),
            pl.BlockSpec((H, I), lambda g: (0, 0)),
        ],
        out_specs=pl.BlockSpec((TILE, H), lambda g: (g, 0)),
        out_shape=jax.ShapeDtypeStruct((T, H), jnp.float32),
        compiler_params=pltpu.CompilerParams(
            dimension_semantics=("arbitrary",)),
        interpret=_INTERPRET,
    )(x_flat, shared_gate_w, shared_up_w, shared_down_w)

    grid_spec = pltpu.PrefetchScalarGridSpec(
        num_scalar_prefetch=1,
        grid=(G_R,),
        in_specs=[
            pl.BlockSpec((TILE, H), lambda g, te_r: (g, 0)),
            pl.BlockSpec((1, 2 * I, H), lambda g, te_r: (te_r[g], 0, 0)),
            pl.BlockSpec((1, H, I), lambda g, te_r: (te_r[g], 0, 0)),
        ],
        out_specs=pl.BlockSpec((TILE, H), lambda g, te_r: (g, 0)),
    )
    y_r = pl.pallas_call(
        _mlp_kernel,
        grid_spec=grid_spec,
        out_shape=jax.ShapeDtypeStruct((NPAD, H), jnp.float32),
        compiler_params=pltpu.CompilerParams(
            dimension_semantics=("arbitrary",)),
        interpret=_INTERPRET,
    )(te, buffer, experts_gate_up, experts_down)

    out = pl.kernel(
        _combine_kernel,
        out_type=jax.ShapeDtypeStruct((T, H), jnp.float32),
        mesh=mesh,
        scratch_types=[
            pltpu.VMEM((_NCHUNK, _CH), jnp.int32),
            pltpu.VMEM((_NCHUNK, _CH), jnp.int32),
            pltpu.VMEM((_TPW, 16), jnp.float32),
            pltpu.VMEM((_TPW, 16), jnp.float32),
            pltpu.VMEM((_CH, H), jnp.float32),
            pltpu.VMEM((_CH, H), jnp.float32),
            pltpu.VMEM((_CH, H), jnp.float32),
            pltpu.VMEM((_CH, H), jnp.float32),
            pltpu.VMEM((_CH, H), jnp.float32),
            pltpu.VMEM((_CH, H), jnp.float32),
            pltpu.SemaphoreType.DMA,
            pltpu.SemaphoreType.DMA,
            pltpu.SemaphoreType.DMA,
            pltpu.SemaphoreType.DMA,
        ],
        interpret=_INTERPRET,
    )(y_s, y_r, p0, p1, w0b, w1b)
    return out.reshape(B, S, Hd)


# speedup vs baseline: 1.0620x; 1.0620x over previous
import functools

import jax
import jax.numpy as jnp
from jax import lax
from jax.experimental import pallas as pl
from jax.experimental.pallas import tpu as pltpu
from jax.experimental.pallas import tpu_sc as plsc

_INTERPRET = False

H = 1024
I = 512
E = 8
T = 2048
TOPK = 2
TILE = 256
G_R = 23
NPAD = G_R * TILE
G_S = T // TILE
GPAD = 128

_NC = 2
_NS = 16
_NW = _NC * _NS
_TPW = T // _NW
_CH = 16
_NCHUNK = _TPW // _CH


def _router_kernel(x_ref, gw_ref, b_ref, w0_ref, w1_ref,
                   pos_ref, te_ref):
    x = x_ref[...]
    gw = gw_ref[...]
    logits = lax.dot_general(x, gw, (((1,), (1,)), ((), ())),
                             preferred_element_type=jnp.float32)
    scores = jax.nn.sigmoid(logits)
    s4c = scores + b_ref[...]
    lane = lax.broadcasted_iota(jnp.int32, (T, E), 1)
    m1 = jnp.max(s4c, axis=1, keepdims=True)
    a1 = jnp.min(jnp.where(s4c >= m1, lane, E), axis=1, keepdims=True)
    oh1 = lane == a1
    s_m = jnp.where(oh1, -jnp.inf, s4c)
    m2 = jnp.max(s_m, axis=1, keepdims=True)
    a2 = jnp.min(jnp.where(s_m >= m2, lane, E), axis=1, keepdims=True)
    oh2 = lane == a2
    w1 = jnp.sum(jnp.where(oh1, scores, 0.0), axis=1, keepdims=True)
    w2 = jnp.sum(jnp.where(oh2, scores, 0.0), axis=1, keepdims=True)
    ws = w1 + w2 + 1e-20
    w0_ref[...] = jnp.broadcast_to(w1 / ws, (T, 16))
    w1_ref[...] = jnp.broadcast_to(w2 / ws, (T, 16))

    cnt = oh1.astype(jnp.float32) + oh2.astype(jnp.float32)
    s = cnt
    sh = 1
    while sh < T:
        s = s + jnp.concatenate(
            [jnp.zeros((sh, E), jnp.float32), s[:-sh, :]], axis=0)
        sh *= 2
    excl = s - cnt
    r0 = jnp.sum(jnp.where(oh1, excl, 0.0), axis=1, keepdims=True)
    r1 = jnp.sum(jnp.where(oh2, excl, 0.0), axis=1, keepdims=True)
    counts = s[T - 1:T, :]
    nt = jnp.floor((counts + (TILE - 1)) / TILE)
    p = nt
    for k in (1, 2, 4):
        p = p + jnp.concatenate(
            [jnp.zeros((1, k), jnp.float32), p[:, :-k]], axis=1)
    tile_start = p - nt
    row_off = tile_start * TILE
    base0 = jnp.sum(jnp.where(oh1, row_off, 0.0), axis=1, keepdims=True)
    base1 = jnp.sum(jnp.where(oh2, row_off, 0.0), axis=1, keepdims=True)
    pos_ref[...] = jnp.concatenate(
        [base0 + r0, base1 + r1], axis=1).astype(jnp.int32)

    gi = lax.broadcasted_iota(jnp.int32, (GPAD, E), 0)
    ts = jnp.floor(tile_start).astype(jnp.int32)
    te = jnp.sum((gi >= ts).astype(jnp.int32), axis=1, keepdims=True) - 1
    te = jnp.maximum(te, 0)
    ntot = jnp.floor(p[:, E - 1:E]).astype(jnp.int32)
    gcol = lax.broadcasted_iota(jnp.int32, (GPAD, 1), 0)
    te_ref[...] = jnp.where(gcol == G_R, ntot, te)


def _dispatch_kernel(x_hbm, p0_hbm, p1_hbm, buf_hbm,
                     idx0_v, idx1_v, rows0_v, rows1_v,
                     semL0, semL1, semS0, semS1):
    wid = lax.axis_index("s") * _NC + lax.axis_index("c")
    pltpu.sync_copy(p0_hbm.at[pl.ds(_NCHUNK * wid, _NCHUNK)], idx0_v)
    pltpu.sync_copy(p1_hbm.at[pl.ds(_NCHUNK * wid, _NCHUNK)], idx1_v)
    rows = (rows0_v, rows1_v)
    semL = (semL0, semL1)
    semS = (semS0, semS1)
    loads = [None, None]
    scats = [None, None]

    def start_load(c):
        base = wid * _TPW + c * _CH
        loads[c % 2] = pltpu.async_copy(
            x_hbm.at[pl.ds(base, _CH)], rows[c % 2], semL[c % 2])

    start_load(0)
    for c in range(_NCHUNK):
        loads[c % 2].wait()
        if c + 1 < _NCHUNK:
            if scats[(c + 1) % 2] is not None:
                for s in scats[(c + 1) % 2]:
                    s.wait()
                scats[(c + 1) % 2] = None
            start_load(c + 1)
        scats[c % 2] = (
            pltpu.async_copy(rows[c % 2], buf_hbm.at[idx0_v.at[c]],
                             semS[c % 2]),
            pltpu.async_copy(rows[c % 2], buf_hbm.at[idx1_v.at[c]],
                             semS[c % 2]),
        )
    for pair in scats:
        if pair is not None:
            for s in pair:
                s.wait()


def _combine_kernel(ys_hbm, yr_hbm, p0_hbm, p1_hbm, w0_hbm, w1_hbm,
                    out_hbm,
                    idx0_v, idx1_v, w0_v, w1_v,
                    rowsA0_v, rowsA1_v, rowsB0_v, rowsB1_v,
                    rowsC0_v, rowsC1_v,
                    semG0, semG1, semO0, semO1):
    wid = lax.axis_index("s") * _NC + lax.axis_index("c")
    pltpu.sync_copy(p0_hbm.at[pl.ds(_NCHUNK * wid, _NCHUNK)], idx0_v)
    pltpu.sync_copy(p1_hbm.at[pl.ds(_NCHUNK * wid, _NCHUNK)], idx1_v)
    pltpu.sync_copy(w0_hbm.at[pl.ds(wid * _TPW, _TPW)], w0_v)
    pltpu.sync_copy(w1_hbm.at[pl.ds(wid * _TPW, _TPW)], w1_v)
    rowsA = (rowsA0_v, rowsA1_v)
    rowsB = (rowsB0_v, rowsB1_v)
    rowsC = (rowsC0_v, rowsC1_v)
    semG = (semG0, semG1)
    semO = (semO0, semO1)
    gaths = [None, None]
    stores = [None, None]

    def start_gathers(c):
        base = wid * _TPW + c * _CH
        k = c % 2
        gaths[k] = (
            pltpu.async_copy(yr_hbm.at[idx0_v.at[c]], rowsA[k], semG[k]),
            pltpu.async_copy(yr_hbm.at[idx1_v.at[c]], rowsB[k], semG[k]),
            pltpu.async_copy(ys_hbm.at[pl.ds(base, _CH)], rowsC[k], semG[k]),
        )

    start_gathers(0)
    for c in range(_NCHUNK):
        k = c % 2
        for gd in gaths[k]:
            gd.wait()
        if c + 1 < _NCHUNK:
            if stores[(c + 1) % 2] is not None:
                stores[(c + 1) % 2].wait()
                stores[(c + 1) % 2] = None
            start_gathers(c + 1)
        for r in range(_CH):
            t = c * _CH + r
            g0 = w0_v[t]
            g1 = w1_v[t]

            def add_body(j, carry, r=r, g0=g0, g1=g1, k=k):
                for u in range(8):
                    sl = pl.ds(j * 128 + u * 16, 16)
                    rowsC[k][r, sl] = (rowsC[k][r, sl]
                                       + g0 * rowsA[k][r, sl]
                                       + g1 * rowsB[k][r, sl])
                return carry

            lax.fori_loop(0, H // 128, add_body, 0)
        base = wid * _TPW + c * _CH
        stores[k] = pltpu.async_copy(
            rowsC[k], out_hbm.at[pl.ds(base, _CH)], semO[k])
    for st in stores:
        if st is not None:
            st.wait()


def _mlp_kernel(te_ref, x_ref, gu_ref, dn_ref, y_ref):
    @pl.when(pl.program_id(0) < te_ref[G_R])
    def _active():
        xb = x_ref[...].astype(jnp.bfloat16)
        gu = gu_ref[0].astype(jnp.bfloat16)
        a = lax.dot_general(xb, gu, (((1,), (1,)), ((), ())),
                            preferred_element_type=jnp.float32)
        g = a[:, :I]
        u = a[:, I:]
        h = (g * jax.nn.sigmoid(g)) * u
        dn = dn_ref[0].astype(jnp.bfloat16)
        y = lax.dot_general(h.astype(jnp.bfloat16), dn,
                            (((1,), (1,)), ((), ())),
                            preferred_element_type=jnp.float32)
        y_ref[...] = y


def _shared_kernel(x_ref, gw_ref, uw_ref, dw_ref, y_ref):
    xb = x_ref[...].astype(jnp.bfloat16)
    g = lax.dot_general(xb, gw_ref[...].astype(jnp.bfloat16),
                        (((1,), (1,)), ((), ())),
                        preferred_element_type=jnp.float32)
    u = lax.dot_general(xb, uw_ref[...].astype(jnp.bfloat16),
                        (((1,), (1,)), ((), ())),
                        preferred_element_type=jnp.float32)
    h = (g * jax.nn.sigmoid(g)) * u
    y = lax.dot_general(h.astype(jnp.bfloat16),
                        dw_ref[...].astype(jnp.bfloat16),
                        (((1,), (1,)), ((), ())),
                        preferred_element_type=jnp.float32)
    y_ref[...] = y


def kernel(x, gate_weight, e_score_correction_bias, experts_gate_up,
           experts_down, shared_gate_w, shared_up_w, shared_down_w):
    B, S, Hd = x.shape
    x_flat = x.reshape(-1, Hd).astype(jnp.float32)

    w0b, w1b, pos, te_full = pl.pallas_call(
        _router_kernel,
        out_shape=[
            jax.ShapeDtypeStruct((T, 16), jnp.float32),
            jax.ShapeDtypeStruct((T, 16), jnp.float32),
            jax.ShapeDtypeStruct((T, TOPK), jnp.int32),
            jax.ShapeDtypeStruct((GPAD, 1), jnp.int32),
        ],
        interpret=_INTERPRET,
    )(x_flat, gate_weight.astype(jnp.float32),
      e_score_correction_bias.reshape(1, E).astype(jnp.float32))

    te = te_full[:G_R + 1, 0]

    p0 = pos[:, 0].reshape(T // _CH, _CH)
    p1 = pos[:, 1].reshape(T // _CH, _CH)
    mesh = plsc.VectorSubcoreMesh(core_axis_name="c", subcore_axis_name="s",
                                  num_cores=_NC, num_subcores=_NS)
    buffer = pl.kernel(
        _dispatch_kernel,
        out_type=jax.ShapeDtypeStruct((NPAD, H), jnp.float32),
        mesh=mesh,
        scratch_types=[
            pltpu.VMEM((_NCHUNK, _CH), jnp.int32),
            pltpu.VMEM((_NCHUNK, _CH), jnp.int32),
            pltpu.VMEM((_CH, H), jnp.float32),
            pltpu.VMEM((_CH, H), jnp.float32),
            pltpu.SemaphoreType.DMA,
            pltpu.SemaphoreType.DMA,
            pltpu.SemaphoreType.DMA,
            pltpu.SemaphoreType.DMA,
        ],
        interpret=_INTERPRET,
    )(x_flat, p0, p1)

    y_s = pl.pallas_call(
        _shared_kernel,
        grid=(G_S,),
        in_specs=[
            pl.BlockSpec((TILE, H), lambda g: (g, 0)),
            pl.BlockSpec((I, H), lambda g: (0, 0)),
            pl.BlockSpec((I, H), lambda g: (0, 0)),
            pl.BlockSpec((H, I), lambda g: (0, 0)),
        ],
        out_specs=pl.BlockSpec((TILE, H), lambda g: (g, 0)),
        out_shape=jax.ShapeDtypeStruct((T, H), jnp.float32),
        compiler_params=pltpu.CompilerParams(
            dimension_semantics=("arbitrary",)),
        interpret=_INTERPRET,
    )(x_flat, shared_gate_w, shared_up_w, shared_down_w)

    grid_spec = pltpu.PrefetchScalarGridSpec(
        num_scalar_prefetch=1,
        grid=(G_R,),
        in_specs=[
            pl.BlockSpec((TILE, H), lambda g, te_r: (g, 0)),
            pl.BlockSpec((1, 2 * I, H), lambda g, te_r: (te_r[g], 0, 0)),
            pl.BlockSpec((1, H, I), lambda g, te_r: (te_r[g], 0, 0)),
        ],
        out_specs=pl.BlockSpec((TILE, H), lambda g, te_r: (g, 0)),
    )
    y_r = pl.pallas_call(
        _mlp_kernel,
        grid_spec=grid_spec,
        out_shape=jax.ShapeDtypeStruct((NPAD, H), jnp.float32),
        compiler_params=pltpu.CompilerParams(
            dimension_semantics=("arbitrary",)),
        interpret=_INTERPRET,
    )(te, buffer, experts_gate_up, experts_down)

    out = pl.kernel(
        _combine_kernel,
        out_type=jax.ShapeDtypeStruct((T, H), jnp.float32),
        mesh=mesh,
        scratch_types=[
            pltpu.VMEM((_NCHUNK, _CH), jnp.int32),
            pltpu.VMEM((_NCHUNK, _CH), jnp.int32),
            pltpu.VMEM((_TPW, 16), jnp.float32),
            pltpu.VMEM((_TPW, 16), jnp.float32),
            pltpu.VMEM((_CH, H), jnp.float32),
            pltpu.VMEM((_CH, H), jnp.float32),
            pltpu.VMEM((_CH, H), jnp.float32),
            pltpu.VMEM((_CH, H), jnp.float32),
            pltpu.VMEM((_CH, H), jnp.float32),
            pltpu.VMEM((_CH, H), jnp.float32),
            pltpu.SemaphoreType.DMA,
            pltpu.SemaphoreType.DMA,
            pltpu.SemaphoreType.DMA,
            pltpu.SemaphoreType.DMA,
        ],
        interpret=_INTERPRET,
    )(y_s, y_r, p0, p1, w0b, w1b)
    return out.reshape(B, S, Hd)
